# Initial kernel scaffold; baseline (speedup 1.0000x reference)
import jax
import jax.numpy as jnp
from jax.experimental import pallas as pl


def kernel(insert, delete, table, W, w_samples, v_noise, trand):
    raise NotImplementedError("write your pallas kernel here")


# speedup vs baseline: 1.2155x; 1.2155x over previous
import functools

import jax
import jax.numpy as jnp
from jax import lax
from jax.experimental import pallas as pl
from jax.experimental.pallas import tpu as pltpu
from jax.experimental.pallas import tpu_sc as plsc

B, L, V, D, EDIT = 4096, 200, 1000000, 64, 128
NORM_MAX = 14.0
NORM_EPS = 0.1

_INFO = plsc.get_sparse_core_info()
NC, NS = _INFO.num_cores, _INFO.num_subcores
NW = NC * NS
NTASK = 2 * B
TPW = NTASK // NW
HALF = L // 2


def _sc_pool_kernel(idx_hbm, table_hbm, out_hbm, idx_v, buf_v, outst_v, sems):
    wid = lax.axis_index("s") * NC + lax.axis_index("c")
    t0 = wid * TPW

    pltpu.sync_copy(idx_hbm.at[pl.ds(t0 * 2, TPW * 2)], idx_v)

    def copies(t, slot):
        return (
            pltpu.make_async_copy(
                table_hbm.at[idx_v.at[2 * t]],
                buf_v.at[slot, pl.ds(0, HALF)],
                sems.at[slot],
            ),
            pltpu.make_async_copy(
                table_hbm.at[idx_v.at[2 * t + 1]],
                buf_v.at[slot, pl.ds(HALF, HALF)],
                sems.at[slot],
            ),
        )

    def fire(t, slot):
        for c in copies(t, slot):
            c.start()

    fire(0, 0)

    @pl.loop(0, TPW, step=2)
    def _task_pair(t):
        for b in range(2):
            tt = t + b
            slot = b

            @pl.when(tt + 1 < TPW)
            def _():
                fire(tt + 1, 1 - slot)

            for c in copies(tt, slot):
                c.wait()

            def red(r, accs):
                return tuple(
                    accs[k] + buf_v[slot, r, pl.ds(16 * k, 16)] for k in range(4)
                )

            zero = jnp.zeros((16,), jnp.float32)
            accs = lax.fori_loop(0, L, red, (zero, zero, zero, zero))
            for k in range(4):
                outst_v[pl.ds(tt * D + 16 * k, 16)] = accs[k]

    pltpu.sync_copy(outst_v, out_hbm.at[pl.ds(t0 * D, TPW * D)])


def _sc_pool(idx_all, table):
    mesh = plsc.VectorSubcoreMesh(core_axis_name="c", subcore_axis_name="s")
    kern = functools.partial(
        pl.kernel,
        out_type=jax.ShapeDtypeStruct((NTASK * D,), jnp.float32),
        mesh=mesh,
        compiler_params=pltpu.CompilerParams(use_tc_tiling_on_sc=False),
        scratch_types=[
            pltpu.VMEM((2 * TPW, HALF), jnp.int32),
            pltpu.VMEM((2, L, D), jnp.float32),
            pltpu.VMEM((TPW * D,), jnp.float32),
            pltpu.SemaphoreType.DMA((2,)),
        ],
    )(_sc_pool_kernel)
    return kern(idx_all, table)


def _tc_post_kernel(sums_ref, w_ref, ws_ref, vn_ref, tr_ref, out_ref):
    ins = sums_ref[0:B, :]
    dele = sums_ref[B:2 * B, :]
    wmat = w_ref[...]
    dn = (((1,), (1,)), ((), ()))
    ins_set = lax.dot_general(ins, wmat, dn, preferred_element_type=jnp.float32)
    del_set = lax.dot_general(dele, wmat, dn, preferred_element_type=jnp.float32)
    mu = jnp.concatenate([ins_set, del_set], axis=1)
    v_noise = vn_ref[...]
    munorm = jnp.sqrt(jnp.sum(mu * mu, axis=1, keepdims=True))
    munoise = jnp.clip(munorm, 0.0, NORM_MAX - NORM_EPS) + tr_ref[...]
    w = ws_ref[...]
    muhat = mu / munorm
    rescale = jnp.sum(muhat * v_noise, axis=1, keepdims=True) / jnp.sqrt(
        jnp.sum(muhat * muhat, axis=1, keepdims=True))
    ortho = v_noise - muhat * rescale
    v = ortho / jnp.sqrt(jnp.sum(ortho * ortho, axis=1, keepdims=True))
    scale_factr = jnp.sqrt(1.0 - jnp.square(w))
    out_ref[...] = (v * scale_factr + mu * w / munorm) * munoise


def _tc_post(sums, W, w_samples, v_noise, trand):
    return pl.pallas_call(
        _tc_post_kernel,
        out_shape=jax.ShapeDtypeStruct((B, EDIT), jnp.float32),
    )(sums, W, w_samples.reshape(B, 1), v_noise, trand)


def kernel(insert, delete, table, W, w_samples, v_noise, trand):
    idx_all = jnp.concatenate(
        [insert.astype(jnp.int32), delete.astype(jnp.int32)], axis=0
    ).reshape(NTASK * 2, HALF)
    sums = _sc_pool(idx_all, table).reshape(NTASK, D)
    return _tc_post(sums, W, w_samples, v_noise, trand)
